# baseline (speedup 1.0000x reference)
import functools

import jax
import jax.numpy as jnp
from jax import lax
from jax.experimental import pallas as pl
from jax.experimental.pallas import tpu as pltpu
from jax.experimental.pallas import tpu_sc as plsc

B = 16384
NS, L = 16, 16
ROWS = B // NS
HALF = ROWS // 2
OUT_W = ROWS * 4

_mesh = plsc.VectorSubcoreMesh(
    core_axis_name="c", subcore_axis_name="s", num_cores=1
)


@functools.partial(
    pl.kernel,
    out_type=jax.ShapeDtypeStruct((B * 4,), jnp.float32),
    mesh=_mesh,
    scratch_types=[
        pltpu.VMEM((ROWS,), jnp.float32),
        pltpu.VMEM((OUT_W,), jnp.float32),
        pltpu.SemaphoreType.DMA,
        pltpu.SemaphoreType.DMA,
        pltpu.SemaphoreType.DMA,
        pltpu.SemaphoreType.DMA,
    ],
    compiler_params=pltpu.CompilerParams(
        needs_layout_passes=False, skip_device_barrier=True
    ),
)
def _build_tril(vec_hbm, out_hbm, v_vmem, o_vmem, si0, si1, so0, so1):
    sid = lax.axis_index("s")
    row0 = sid * ROWS
    cp_in = []
    for h, sem in ((0, si0), (1, si1)):
        cp_in.append(
            pltpu.async_copy(
                vec_hbm.at[pl.ds(row0 + h * HALF, HALF)],
                v_vmem.at[pl.ds(h * HALF, HALF)],
                sem,
            )
        )
    lane = lax.iota(jnp.int32, L)
    m = lane % 4
    const = jnp.where((m == 0) | (m == 3), 1.0, 0.0).astype(jnp.float32)
    cp_out = []
    for h, sem in ((0, so0), (1, so1)):
        cp_in[h].wait()
        for g in range(h * HALF // L, (h + 1) * HALF // L):
            base = g * 4 * L
            for k in range(4):
                o_vmem[pl.ds(base + k * L, L)] = const
            v = v_vmem[pl.ds(g * L, L)]
            plsc.store_scatter(o_vmem, [lane * 4 + (base + 2)], v)
        cp_out.append(
            pltpu.async_copy(
                o_vmem.at[pl.ds(h * HALF * 4, HALF * 4)],
                out_hbm.at[pl.ds(row0 * 4 + h * HALF * 4, HALF * 4)],
                sem,
            )
        )
    cp_out[0].wait()
    cp_out[1].wait()


def kernel(vec):
    flat = _build_tril(vec.reshape(-1))
    return flat.reshape(B, 2, 2)


# speedup vs baseline: 1.0048x; 1.0048x over previous
import functools

import jax
import jax.numpy as jnp
from jax import lax
from jax.experimental import pallas as pl
from jax.experimental.pallas import tpu as pltpu
from jax.experimental.pallas import tpu_sc as plsc

B = 16384
NS, L = 16, 16
ROWS = B // NS
OUT_W = ROWS * 4

_mesh = plsc.VectorSubcoreMesh(
    core_axis_name="c", subcore_axis_name="s", num_cores=1
)


@functools.partial(
    pl.kernel,
    out_type=jax.ShapeDtypeStruct((B * 4,), jnp.float32),
    mesh=_mesh,
    scratch_types=[
        pltpu.VMEM((ROWS,), jnp.float32),
        pltpu.VMEM((OUT_W,), jnp.float32),
    ],
    compiler_params=pltpu.CompilerParams(
        needs_layout_passes=False, skip_device_barrier=True
    ),
)
def _build_tril(vec_hbm, out_hbm, v_vmem, o_vmem):
    sid = lax.axis_index("s")
    pltpu.sync_copy(vec_hbm.at[pl.ds(sid * ROWS, ROWS)], v_vmem)
    lane = lax.iota(jnp.int32, L)
    m = lane % 4
    const = jnp.where((m == 0) | (m == 3), 1.0, 0.0).astype(jnp.float32)

    def group(g, carry):
        base = g * (4 * L)
        for k in range(4):
            o_vmem[pl.ds(base + k * L, L)] = const
        v = v_vmem[pl.ds(g * L, L)]
        plsc.store_scatter(o_vmem, [lane * 4 + (base + 2)], v)
        return carry

    lax.fori_loop(0, ROWS // L, group, 0)
    pltpu.sync_copy(o_vmem, out_hbm.at[pl.ds(sid * OUT_W, OUT_W)])


def kernel(vec):
    flat = _build_tril(vec.reshape(-1))
    return flat.reshape(B, 2, 2)


# speedup vs baseline: 1.0092x; 1.0044x over previous
import functools

import jax
import jax.numpy as jnp
from jax import lax
from jax.experimental import pallas as pl
from jax.experimental.pallas import tpu as pltpu
from jax.experimental.pallas import tpu_sc as plsc

B = 16384
NS, L = 16, 16
ROWS = B // NS
OUT_W = ROWS * 4

_mesh = plsc.VectorSubcoreMesh(
    core_axis_name="c", subcore_axis_name="s", num_cores=1
)


@functools.partial(
    pl.kernel,
    out_type=jax.ShapeDtypeStruct((B * 4,), jnp.float32),
    mesh=_mesh,
    scratch_types=[
        pltpu.VMEM((ROWS,), jnp.float32),
        pltpu.VMEM((OUT_W,), jnp.float32),
    ],
    compiler_params=pltpu.CompilerParams(
        needs_layout_passes=False, skip_device_barrier=True
    ),
)
def _build_tril(vec_hbm, out_hbm, v_vmem, o_vmem):
    sid = lax.axis_index("s")
    pltpu.sync_copy(vec_hbm.at[pl.ds(sid * ROWS, ROWS)], v_vmem)
    lane = lax.iota(jnp.int32, L)
    m = lane % 4
    const = jnp.where((m == 0) | (m == 3), 1.0, 0.0).astype(jnp.float32)

    @plsc.parallel_loop(0, ROWS // L, unroll=2)
    def group(g):
        base = g * (4 * L)
        for k in range(4):
            o_vmem[pl.ds(base + k * L, L)] = const
        v = v_vmem[pl.ds(g * L, L)]
        plsc.store_scatter(o_vmem, [lane * 4 + (base + 2)], v)
    pltpu.sync_copy(o_vmem, out_hbm.at[pl.ds(sid * OUT_W, OUT_W)])


def kernel(vec):
    flat = _build_tril(vec.reshape(-1))
    return flat.reshape(B, 2, 2)
